# baseline (speedup 1.0000x reference)
import functools

import jax
import jax.numpy as jnp
from jax import lax
from jax.experimental import pallas as pl
from jax.experimental.pallas import tpu as pltpu
from jax.experimental.pallas import tpu_sc as plsc

_NE = 1024
_D = 64
_BLK = 9216
_COMMIT = 0.25

_NW = 32
_S = 1


def _vq_tc_body(x_ref, w_ref, idx_ref, acc_ref, wpad_ref, w2_ref):
    i = pl.program_id(0)

    @pl.when(i == 0)
    def _():
        w0 = w_ref[...]
        w2_ref[...] = jnp.sum(w0 * w0, axis=1)[None, :]
        acc_ref[...] = jnp.zeros_like(acc_ref)
        wpad_ref[...] = jnp.pad(w0, ((0, 0), (0, 128 - _D)))

    x = x_ref[...]
    x2 = jnp.sum(x * x, axis=1, keepdims=True)
    m = lax.dot_general(x, w_ref[...], (((1,), (1,)), ((), ())))
    d = (x2 + w2_ref[...]) - 2.0 * m
    dmin = jnp.min(d, axis=1, keepdims=True)
    col = lax.broadcasted_iota(jnp.int32, (1, _NE), 1).astype(jnp.float32)
    idx = jnp.min(jnp.where(d == dmin, col, float(_NE)), axis=1)
    idx_ref[...] = idx.astype(jnp.int32)[None, None, :]

    s = jnp.sum(dmin)
    acc_ref[...] += jnp.full((8, 128), s * (1.0 / 1024.0), jnp.float32)


def _sc_gather_body(w_hbm, idx_hbm, out_hbm, idx_v, rows_v, gsem, osem):
    wid = lax.axis_index("s") * 2 + lax.axis_index("c")
    nchunk = idx_v.shape[0]
    nbuf = rows_v.shape[0]
    chunk = idx_v.shape[1]
    pltpu.sync_copy(idx_hbm.at[wid], idx_v)
    gath = [pltpu.async_copy(w_hbm.at[idx_v.at[j]], rows_v.at[j], gsem)
            for j in range(min(nbuf, nchunk))]
    outc = [None] * nbuf
    for j in range(nchunk):
        b = j % nbuf
        gath[b].wait()
        outc[b] = pltpu.async_copy(
            rows_v.at[b], out_hbm.at[wid].at[pl.ds(j * chunk, chunk)], osem)
        nxt = j + nbuf
        if nxt < nchunk:
            outc[b].wait()
            gath[b] = pltpu.async_copy(
                w_hbm.at[idx_v.at[nxt]], rows_v.at[nxt % nbuf], gsem)
    for j in range(max(0, nchunk - nbuf), nchunk):
        outc[j % nbuf].wait()


def kernel(inputs, weight):
    x = inputs.reshape(-1, _D)
    n = x.shape[0]
    npart = n // _S
    nb = npart // _BLK
    b_per_w = npart // _NW
    chunk = max(c for c in range(8, 129, 8) if b_per_w % c == 0)
    nchunk = b_per_w // chunk
    nbuf = min(4, nchunk)
    mesh = plsc.VectorSubcoreMesh(core_axis_name="c", subcore_axis_name="s")

    idxs, qs, accs = [], [], []
    for p in range(_S):
        idx3, acc, w_pad = pl.pallas_call(
            _vq_tc_body,
            grid=(nb,),
            in_specs=[pl.BlockSpec((_BLK, _D), lambda i, p=p: (p * nb + i, 0)),
                      pl.BlockSpec((_NE, _D), lambda i: (0, 0))],
            out_specs=[pl.BlockSpec((1, 1, _BLK), lambda i: (i, 0, 0)),
                       pl.BlockSpec((8, 128), lambda i: (0, 0)),
                       pl.BlockSpec((_NE, 128), lambda i: (0, 0))],
            out_shape=[jax.ShapeDtypeStruct((nb, 1, _BLK), jnp.int32),
                       jax.ShapeDtypeStruct((8, 128), jnp.float32),
                       jax.ShapeDtypeStruct((_NE, 128), jnp.float32)],
            scratch_shapes=[pltpu.VMEM((1, _NE), jnp.float32)],
        )(x, weight)
        idx_sc = idx3.reshape(_NW, nchunk, chunk)
        q = pl.kernel(
            _sc_gather_body,
            out_type=jax.ShapeDtypeStruct((_NW, b_per_w, 128), jnp.float32),
            mesh=mesh,
            scratch_types=[
                pltpu.VMEM((nchunk, chunk), jnp.int32),
                pltpu.VMEM((nbuf, chunk, 128), jnp.float32),
                pltpu.SemaphoreType.DMA,
                pltpu.SemaphoreType.DMA,
            ],
        )(w_pad, idx_sc)
        idxs.append(idx3)
        qs.append(q.reshape(npart, 128))
        accs.append(acc)

    e = sum(jnp.sum(a) for a in accs) / (n * _D)
    loss = e + _COMMIT * e
    q64 = jnp.concatenate(qs, axis=0)[:, :_D]
    idx_full = jnp.concatenate(idxs, axis=0)
    return loss, q64.reshape(inputs.shape), idx_full.reshape(inputs.shape[:-1])


# speedup vs baseline: 1.0764x; 1.0764x over previous
import functools

import jax
import jax.numpy as jnp
from jax import lax
from jax.experimental import pallas as pl
from jax.experimental.pallas import tpu as pltpu
from jax.experimental.pallas import tpu_sc as plsc

_NE = 1024
_D = 64
_BLK = 9216
_COMMIT = 0.25

_NW = 32
_S = 1


def _vq_tc_body(x_ref, w_ref, idx_ref, acc_ref, wpad_ref, w2_ref):
    i = pl.program_id(0)

    @pl.when(i == 0)
    def _():
        w0 = w_ref[...]
        w2_ref[...] = jnp.sum(w0 * w0, axis=1)[None, :]
        acc_ref[...] = jnp.zeros_like(acc_ref)
        wpad_ref[...] = jnp.pad(w0, ((0, 0), (0, 128 - _D)))

    x = x_ref[...]
    x2 = jnp.sum(x * x, axis=1, keepdims=True)
    m = lax.dot_general(x, w_ref[...], (((1,), (1,)), ((), ())))
    d = (x2 + w2_ref[...]) - 2.0 * m
    dmin = jnp.min(d, axis=1, keepdims=True)
    col = lax.broadcasted_iota(jnp.int32, (1, _NE), 1)
    idx = jnp.min(jnp.where(d == dmin, col, _NE), axis=1)
    idx_ref[...] = idx[None, None, :]

    s = jnp.sum(dmin)
    acc_ref[...] += jnp.full((8, 128), s * (1.0 / 1024.0), jnp.float32)


def _sc_gather_body(w_hbm, idx_hbm, out_hbm, idx_v, rows_v, gsem, osem):
    wid = lax.axis_index("s") * 2 + lax.axis_index("c")
    nchunk = idx_v.shape[0]
    nbuf = rows_v.shape[0]
    chunk = idx_v.shape[1]
    pltpu.sync_copy(idx_hbm.at[wid], idx_v)
    gath = [pltpu.async_copy(w_hbm.at[idx_v.at[j]], rows_v.at[j], gsem)
            for j in range(min(nbuf, nchunk))]
    outc = [None] * nbuf
    for j in range(nchunk):
        b = j % nbuf
        gath[b].wait()
        outc[b] = pltpu.async_copy(
            rows_v.at[b], out_hbm.at[wid].at[pl.ds(j * chunk, chunk)], osem)
        nxt = j + nbuf
        if nxt < nchunk:
            outc[b].wait()
            gath[b] = pltpu.async_copy(
                w_hbm.at[idx_v.at[nxt]], rows_v.at[nxt % nbuf], gsem)
    for j in range(max(0, nchunk - nbuf), nchunk):
        outc[j % nbuf].wait()


def kernel(inputs, weight):
    x = inputs.reshape(-1, _D)
    n = x.shape[0]
    npart = n // _S
    nb = npart // _BLK
    b_per_w = npart // _NW
    chunk = max(c for c in range(8, 129, 8) if b_per_w % c == 0)
    nchunk = b_per_w // chunk
    nbuf = min(4, nchunk)
    mesh = plsc.VectorSubcoreMesh(core_axis_name="c", subcore_axis_name="s")

    idxs, qs, accs = [], [], []
    for p in range(_S):
        idx3, acc, w_pad = pl.pallas_call(
            _vq_tc_body,
            grid=(nb,),
            in_specs=[pl.BlockSpec((_BLK, _D), lambda i, p=p: (p * nb + i, 0)),
                      pl.BlockSpec((_NE, _D), lambda i: (0, 0))],
            out_specs=[pl.BlockSpec((1, 1, _BLK), lambda i: (i, 0, 0)),
                       pl.BlockSpec((8, 128), lambda i: (0, 0)),
                       pl.BlockSpec((_NE, 128), lambda i: (0, 0))],
            out_shape=[jax.ShapeDtypeStruct((nb, 1, _BLK), jnp.int32),
                       jax.ShapeDtypeStruct((8, 128), jnp.float32),
                       jax.ShapeDtypeStruct((_NE, 128), jnp.float32)],
            scratch_shapes=[pltpu.VMEM((1, _NE), jnp.float32)],
        )(x, weight)
        idx_sc = idx3.reshape(_NW, nchunk, chunk)
        q = pl.kernel(
            _sc_gather_body,
            out_type=jax.ShapeDtypeStruct((_NW, b_per_w, 128), jnp.float32),
            mesh=mesh,
            scratch_types=[
                pltpu.VMEM((nchunk, chunk), jnp.int32),
                pltpu.VMEM((nbuf, chunk, 128), jnp.float32),
                pltpu.SemaphoreType.DMA,
                pltpu.SemaphoreType.DMA,
            ],
        )(w_pad, idx_sc)
        idxs.append(idx3)
        qs.append(q.reshape(npart, 128))
        accs.append(acc)

    e = sum(jnp.sum(a) for a in accs) / (n * _D)
    loss = e + _COMMIT * e
    q64 = jnp.concatenate(qs, axis=0)[:, :_D]
    idx_full = jnp.concatenate(idxs, axis=0)
    return loss, q64.reshape(inputs.shape), idx_full.reshape(inputs.shape[:-1])
